# baseline (speedup 1.0000x reference)
import jax
import jax.numpy as jnp
from jax import lax
from jax.experimental import pallas as pl
from jax.experimental.pallas import tpu as pltpu
from jax.experimental.pallas import tpu_sc as plsc

N_NODES = 100000
IN_C = 100000
HID_C = 32
OUT_C = 64
N_USER = 30000

NC = 2
NS = 16
L = 16

C = 512
CI = C // 128

E_REAL = 1600000
G_CHUNKS = E_REAL // C

RPT = -(-N_NODES // NS // 8) * 8
N_PAD = NS * RPT

_GDN = lax.GatherDimensionNumbers(
    offset_dims=(), collapsed_slice_dims=(0,), start_index_map=(0,))


def _splat(vec, j):
    idx = jnp.full((L, 1), j, jnp.int32)
    return lax.gather(vec, idx, _GDN, (1,),
                      mode=lax.GatherScatterMode.PROMISE_IN_BOUNDS)


def _sc_body(table_ref, src_ref, et_ref, dst_ref, norm_ref, out_ref,
             src4, et4, gidx4, dstc4, norm4, rows_v, acc, edg, gth, sct):
    h = lax.axis_index("c")
    t = lax.axis_index("s")

    def _zrow(i, _):
        rows_v[0, i, :] = jnp.zeros((L,), jnp.float32)
        return 0
    lax.fori_loop(0, C, _zrow, 0)
    r0 = t * RPT
    for i in range(RPT // C):
        pltpu.sync_copy(rows_v.at[0], acc.at[pl.ds(r0 + i * C, C)])
    rem = RPT - (RPT // C) * C
    if rem:
        pltpu.sync_copy(rows_v.at[0, pl.ds(0, rem)],
                        acc.at[pl.ds(r0 + (RPT // C) * C, rem)])
    plsc.subcore_barrier()

    n_t = (G_CHUNKS - 1 - t) // NS + 1

    def _fire_edge(ci, b):
        gc = ci * NS + t
        base = pl.multiple_of(gc * C, C)
        rbase = pl.multiple_of(gc * CI, CI)
        pltpu.async_copy(src_ref.at[pl.ds(base, C)], src4.at[b], edg.at[b])
        pltpu.async_copy(et_ref.at[pl.ds(base, C)], et4.at[b], edg.at[b])
        pltpu.async_copy(norm_ref.at[pl.ds(base, C)], norm4.at[b], edg.at[b])
        pltpu.async_copy(dst_ref.at[pl.ds(rbase, CI)], dstc4.at[b], edg.at[b])

    def _wait_edge(b):
        pltpu.make_async_copy(src_ref.at[pl.ds(0, C)], src4.at[b], edg.at[b]).wait()
        pltpu.make_async_copy(et_ref.at[pl.ds(0, C)], et4.at[b], edg.at[b]).wait()
        pltpu.make_async_copy(norm_ref.at[pl.ds(0, C)], norm4.at[b], edg.at[b]).wait()
        pltpu.make_async_copy(dst_ref.at[pl.ds(0, CI)], dstc4.at[b], edg.at[b]).wait()

    def _scale_fire_scatter(ci):
        pr = lax.bitwise_and(ci - 1, 1)
        pb = lax.bitwise_and(ci - 1, 3)
        pltpu.make_async_copy(table_ref.at[pl.ds(0, C)], rows_v.at[pr],
                              gth.at[pr]).wait()

        @plsc.parallel_loop(0, C // L, 1, unroll=2)
        def _scale(g):
            nv = norm4[pb, pl.ds(g * L, L)]
            rr = g * L
            for j in range(L):
                rows_v[pr, rr + j, :] = rows_v[pr, rr + j, :] * _splat(nv, j)

        for j in range(CI):
            pltpu.async_copy(rows_v.at[pr, pl.ds(j * 128, 128)],
                             acc.at[dstc4.at[pb, j]], sct.at[pr], add=True)

    def _chunk(ci, _):
        b = lax.bitwise_and(ci, 3)
        r = lax.bitwise_and(ci, 1)

        pl.when(ci >= 1)(lambda: _scale_fire_scatter(ci))

        _wait_edge(b)

        def _gidx(g, _):
            s = src4[b, pl.ds(g * L, L)]
            e = et4[b, pl.ds(g * L, L)]
            gidx4[b, pl.ds(g * L, L)] = (e * IN_C + s) * 2 + h
            return 0
        lax.fori_loop(0, C // L, _gidx, 0)

        def _wait_sct():
            pltpu.make_async_copy(table_ref.at[pl.ds(0, C)], rows_v.at[r],
                                  sct.at[r]).wait()
        pl.when(ci >= 2)(_wait_sct)

        for j in range(CI):
            pltpu.async_copy(table_ref.at[gidx4.at[b, pl.ds(j * 128, 128)]],
                             rows_v.at[r, pl.ds(j * 128, 128)], gth.at[r])

        nb = lax.bitwise_and(ci + 1, 3)
        pl.when(ci + 1 < n_t)(lambda: _fire_edge(ci + 1, nb))
        return 0

    _fire_edge(0, 0)
    lax.fori_loop(0, n_t, _chunk, 0)
    _scale_fire_scatter(n_t)
    for rr in range(2):
        pltpu.make_async_copy(table_ref.at[pl.ds(0, C)], rows_v.at[rr],
                              sct.at[rr]).wait()
    plsc.subcore_barrier()

    for i in range(RPT // C):
        pltpu.sync_copy(acc.at[pl.ds(r0 + i * C, C)],
                        out_ref.at[h, pl.ds(r0 + i * C, C)])
    if rem:
        pltpu.sync_copy(acc.at[pl.ds(r0 + (RPT // C) * C, rem)],
                        out_ref.at[h, pl.ds(r0 + (RPT // C) * C, rem)])


def _sc_aggregate(table2, src, et, dst2, norm):
    fn = pl.kernel(
        _sc_body,
        out_type=jax.ShapeDtypeStruct((NC, N_PAD, L), jnp.float32),
        mesh=plsc.VectorSubcoreMesh(core_axis_name="c", subcore_axis_name="s"),
        compiler_params=pltpu.CompilerParams(use_tc_tiling_on_sc=False),
        scratch_types=[
            pltpu.VMEM((4, C), jnp.int32),
            pltpu.VMEM((4, C), jnp.int32),
            pltpu.VMEM((4, C), jnp.int32),
            pltpu.VMEM((4, CI, 128), jnp.int32),
            pltpu.VMEM((4, C), jnp.float32),
            pltpu.VMEM((2, C, L), jnp.float32),
            pltpu.VMEM_SHARED((N_PAD, L), jnp.float32),
            pltpu.SemaphoreType.DMA((4,)),
            pltpu.SemaphoreType.DMA((2,)),
            pltpu.SemaphoreType.DMA((2,)),
        ],
    )
    return fn(table2, src, et, dst2, norm)


def _tc_body(a_ref, b_ref, w_ref, o_ref):
    a = jnp.maximum(a_ref[0], 0.0)
    b = jnp.maximum(b_ref[0], 0.0)
    w = w_ref[...]
    o = lax.dot_general(a, w[0:16, :], (((1,), (0,)), ((), ())),
                        preferred_element_type=jnp.float32)
    o = o + lax.dot_general(b, w[16:32, :], (((1,), (0,)), ((), ())),
                            preferred_element_type=jnp.float32)
    o_ref[...] = jnp.maximum(o, 0.0)


def _tc_dense(agg, w, n_rows, row_off):
    R = 2000
    grid = n_rows // R
    off = row_off // R
    return pl.pallas_call(
        _tc_body,
        grid=(grid,),
        in_specs=[
            pl.BlockSpec((1, R, L), lambda i: (0, i + off, 0)),
            pl.BlockSpec((1, R, L), lambda i: (1, i + off, 0)),
            pl.BlockSpec((HID_C, OUT_C), lambda i: (0, 0)),
        ],
        out_specs=pl.BlockSpec((R, OUT_C), lambda i: (i, 0)),
        out_shape=jax.ShapeDtypeStruct((n_rows, OUT_C), jnp.float32),
    )(agg, agg, w)


def kernel(x, edge_index, edge_type, edge_norm, rgc_weight, dense_w_u, dense_w_i):
    src = edge_index[0].astype(jnp.int32)
    et = edge_type.astype(jnp.int32)
    norm = edge_norm.astype(jnp.float32)
    dst2 = edge_index[1].astype(jnp.int32).reshape(E_REAL // 128, 128)
    table2 = rgc_weight.reshape(-1, L)

    agg = _sc_aggregate(table2, src, et, dst2, norm)

    u_out = _tc_dense(agg, dense_w_u, N_USER, 0)
    i_out = _tc_dense(agg, dense_w_i, N_NODES - N_USER, N_USER)
    return (u_out, i_out)


# speedup vs baseline: 1.0487x; 1.0487x over previous
import jax
import jax.numpy as jnp
from jax import lax
from jax.experimental import pallas as pl
from jax.experimental.pallas import tpu as pltpu
from jax.experimental.pallas import tpu_sc as plsc

N_NODES = 100000
IN_C = 100000
HID_C = 32
OUT_C = 64
N_USER = 30000

NC = 2
NS = 16
L = 16

C = 512
CI = C // 128

E_REAL = 1600000
G_CHUNKS = E_REAL // C

RPT = -(-N_NODES // NS // 8) * 8
N_PAD = NS * RPT

_GDN = lax.GatherDimensionNumbers(
    offset_dims=(), collapsed_slice_dims=(0,), start_index_map=(0,))


def _splat(vec, j):
    idx = jnp.full((L, 1), j, jnp.int32)
    return lax.gather(vec, idx, _GDN, (1,),
                      mode=lax.GatherScatterMode.PROMISE_IN_BOUNDS)


def _sc_body(table_ref, src_ref, et_ref, dst_ref, norm_ref, out_ref,
             src4, et4, gidx4, dstc4, norm4, rows_v, acc, edg, gth, sct):
    h = lax.axis_index("c")
    t = lax.axis_index("s")

    def _zrow(i, _):
        rows_v[0, i, :] = jnp.zeros((L,), jnp.float32)
        return 0
    lax.fori_loop(0, C, _zrow, 0)
    r0 = t * RPT
    for i in range(RPT // C):
        pltpu.sync_copy(rows_v.at[0], acc.at[pl.ds(r0 + i * C, C)])
    rem = RPT - (RPT // C) * C
    if rem:
        pltpu.sync_copy(rows_v.at[0, pl.ds(0, rem)],
                        acc.at[pl.ds(r0 + (RPT // C) * C, rem)])
    plsc.subcore_barrier()

    n_t = (G_CHUNKS - 1 - t) // NS + 1

    def _fire_edge(ci, b):
        gc = ci * NS + t
        base = pl.multiple_of(gc * C, C)
        rbase = pl.multiple_of(gc * CI, CI)
        pltpu.async_copy(src_ref.at[pl.ds(base, C)], src4.at[b], edg.at[b])
        pltpu.async_copy(et_ref.at[pl.ds(base, C)], et4.at[b], edg.at[b])
        pltpu.async_copy(norm_ref.at[pl.ds(base, C)], norm4.at[b], edg.at[b])
        pltpu.async_copy(dst_ref.at[pl.ds(rbase, CI)], dstc4.at[b], edg.at[b])

    def _wait_edge(b):
        pltpu.make_async_copy(src_ref.at[pl.ds(0, C)], src4.at[b], edg.at[b]).wait()
        pltpu.make_async_copy(et_ref.at[pl.ds(0, C)], et4.at[b], edg.at[b]).wait()
        pltpu.make_async_copy(norm_ref.at[pl.ds(0, C)], norm4.at[b], edg.at[b]).wait()
        pltpu.make_async_copy(dst_ref.at[pl.ds(0, CI)], dstc4.at[b], edg.at[b]).wait()

    def _scale_fire_scatter(ci):
        pr = lax.bitwise_and(ci - 1, 1)
        pb = lax.bitwise_and(ci - 1, 3)
        pltpu.make_async_copy(table_ref.at[pl.ds(0, C)], rows_v.at[pr],
                              gth.at[pr]).wait()

        @plsc.parallel_loop(0, C // L, 1, unroll=2)
        def _scale(g):
            nv = norm4[pb, pl.ds(g * L, L)]
            rr = g * L
            for j in range(L):
                rows_v[pr, rr + j, :] = rows_v[pr, rr + j, :] * _splat(nv, j)

        for j in range(CI):
            pltpu.async_copy(rows_v.at[pr, pl.ds(j * 128, 128)],
                             acc.at[dstc4.at[pb, j]], sct.at[pr], add=True)

    def _chunk(ci, _):
        b = lax.bitwise_and(ci, 3)
        r = lax.bitwise_and(ci, 1)

        pl.when(ci >= 1)(lambda: _scale_fire_scatter(ci))

        _wait_edge(b)

        def _gidx(g, _):
            s = src4[b, pl.ds(g * L, L)]
            e = et4[b, pl.ds(g * L, L)]
            gidx4[b, pl.ds(g * L, L)] = (e * IN_C + s) * 2 + h
            return 0
        lax.fori_loop(0, C // L, _gidx, 0)

        def _wait_sct():
            pltpu.make_async_copy(table_ref.at[pl.ds(0, C)], rows_v.at[r],
                                  sct.at[r]).wait()
        pl.when(ci >= 2)(_wait_sct)

        for j in range(CI):
            pltpu.async_copy(table_ref.at[gidx4.at[b, pl.ds(j * 128, 128)]],
                             rows_v.at[r, pl.ds(j * 128, 128)], gth.at[r])

        nb = lax.bitwise_and(ci + 1, 3)
        pl.when(ci + 1 < n_t)(lambda: _fire_edge(ci + 1, nb))
        return 0

    _fire_edge(0, 0)
    lax.fori_loop(0, n_t, _chunk, 0)
    _scale_fire_scatter(n_t)
    for rr in range(2):
        pltpu.make_async_copy(table_ref.at[pl.ds(0, C)], rows_v.at[rr],
                              sct.at[rr]).wait()
    plsc.subcore_barrier()

    for i in range(RPT // C):
        pltpu.sync_copy(acc.at[pl.ds(r0 + i * C, C)],
                        out_ref.at[h, pl.ds(r0 + i * C, C)])
    if rem:
        pltpu.sync_copy(acc.at[pl.ds(r0 + (RPT // C) * C, rem)],
                        out_ref.at[h, pl.ds(r0 + (RPT // C) * C, rem)])


def _sc_aggregate(table2, src, et, dst2, norm):
    fn = pl.kernel(
        _sc_body,
        out_type=jax.ShapeDtypeStruct((NC, N_PAD, L), jnp.float32),
        mesh=plsc.VectorSubcoreMesh(core_axis_name="c", subcore_axis_name="s"),
        compiler_params=pltpu.CompilerParams(use_tc_tiling_on_sc=False),
        scratch_types=[
            pltpu.VMEM((4, C), jnp.int32),
            pltpu.VMEM((4, C), jnp.int32),
            pltpu.VMEM((4, C), jnp.int32),
            pltpu.VMEM((4, CI, 128), jnp.int32),
            pltpu.VMEM((4, C), jnp.float32),
            pltpu.VMEM((2, C, L), jnp.float32),
            pltpu.VMEM_SHARED((N_PAD, L), jnp.float32),
            pltpu.SemaphoreType.DMA((4,)),
            pltpu.SemaphoreType.DMA((2,)),
            pltpu.SemaphoreType.DMA((2,)),
        ],
    )
    return fn(table2, src, et, dst2, norm)


def _tc_body(a_ref, b_ref, wu_ref, wi_ref, o_ref):
    i = pl.program_id(0)
    a = jnp.maximum(a_ref[0], 0.0)
    b = jnp.maximum(b_ref[0], 0.0)
    x = jnp.concatenate([a, b], axis=1)
    ou = lax.dot_general(x, wu_ref[...], (((1,), (0,)), ((), ())),
                         preferred_element_type=jnp.float32)
    oi = lax.dot_general(x, wi_ref[...], (((1,), (0,)), ((), ())),
                         preferred_element_type=jnp.float32)
    rows = lax.broadcasted_iota(jnp.int32, (BF, 8 * OUT_C), 0) + i * BF
    o = jnp.where(rows < N_USER // 8, ou, oi)
    o_ref[...] = jnp.maximum(o, 0.0)


BF = 736


def _tc_dense(agg_flat, w_u, w_i):
    eye8 = jnp.eye(8, dtype=jnp.float32)
    wu = jnp.concatenate([jnp.kron(eye8, w_u[:16]), jnp.kron(eye8, w_u[16:])])
    wi = jnp.concatenate([jnp.kron(eye8, w_i[:16]), jnp.kron(eye8, w_i[16:])])
    n_flat = N_PAD // 8
    grid = n_flat // BF
    return pl.pallas_call(
        _tc_body,
        grid=(grid,),
        in_specs=[
            pl.BlockSpec((1, BF, 128), lambda i: (0, i, 0)),
            pl.BlockSpec((1, BF, 128), lambda i: (1, i, 0)),
            pl.BlockSpec((256, 8 * OUT_C), lambda i: (0, 0)),
            pl.BlockSpec((256, 8 * OUT_C), lambda i: (0, 0)),
        ],
        out_specs=pl.BlockSpec((BF, 8 * OUT_C), lambda i: (i, 0)),
        out_shape=jax.ShapeDtypeStruct((n_flat, 8 * OUT_C), jnp.float32),
    )(agg_flat, agg_flat, wu, wi)


def kernel(x, edge_index, edge_type, edge_norm, rgc_weight, dense_w_u, dense_w_i):
    src = edge_index[0].astype(jnp.int32)
    et = edge_type.astype(jnp.int32)
    norm = edge_norm.astype(jnp.float32)
    dst2 = edge_index[1].astype(jnp.int32).reshape(E_REAL // 128, 128)
    table2 = rgc_weight.reshape(-1, L)

    agg = _sc_aggregate(table2, src, et, dst2, norm)
    agg_flat = agg.reshape(NC, N_PAD // 8, 128)

    out = _tc_dense(agg_flat, dense_w_u, dense_w_i)
    out = out.reshape(N_PAD, OUT_C)
    return (out[:N_USER], out[N_USER:N_NODES])


# speedup vs baseline: 1.3218x; 1.2604x over previous
import jax
import jax.numpy as jnp
from jax import lax
from jax.experimental import pallas as pl
from jax.experimental.pallas import tpu as pltpu
from jax.experimental.pallas import tpu_sc as plsc

N_NODES = 100000
IN_C = 100000
HID_C = 32
OUT_C = 64
N_USER = 30000

NC = 2
NS = 16
L = 16

C = 512
CI = C // 128

E_REAL = 1600000
G_CHUNKS = E_REAL // C

RPT = -(-N_NODES // NS // 8) * 8
N_PAD = NS * RPT

_GDN = lax.GatherDimensionNumbers(
    offset_dims=(), collapsed_slice_dims=(0,), start_index_map=(0,))


def _splat(vec, j):
    idx = jnp.full((L, 1), j, jnp.int32)
    return lax.gather(vec, idx, _GDN, (1,),
                      mode=lax.GatherScatterMode.PROMISE_IN_BOUNDS)


def _sc_body(table_ref, src_ref, et_ref, dst_ref, norm_ref, out_ref,
             src4, et4, gidx4, dstc4, norm4, rows_v, acc, edg, gth, sct):
    h = lax.axis_index("c")
    t = lax.axis_index("s")

    def _zrow(i, _):
        rows_v[0, i, :] = jnp.zeros((L,), jnp.float32)
        return 0
    lax.fori_loop(0, C, _zrow, 0)
    r0 = t * RPT
    for i in range(RPT // C):
        pltpu.sync_copy(rows_v.at[0], acc.at[pl.ds(r0 + i * C, C)])
    rem = RPT - (RPT // C) * C
    if rem:
        pltpu.sync_copy(rows_v.at[0, pl.ds(0, rem)],
                        acc.at[pl.ds(r0 + (RPT // C) * C, rem)])
    plsc.subcore_barrier()

    n_t = (G_CHUNKS - 1 - t) // NS + 1

    def _fire_edge(ci, b):
        gc = ci * NS + t
        base = pl.multiple_of(gc * C, C)
        rbase = pl.multiple_of(gc * CI, CI)
        pltpu.async_copy(src_ref.at[pl.ds(base, C)], src4.at[b], edg.at[b])
        pltpu.async_copy(et_ref.at[pl.ds(base, C)], et4.at[b], edg.at[b])
        pltpu.async_copy(norm_ref.at[pl.ds(base, C)], norm4.at[b], edg.at[b])
        pltpu.async_copy(dst_ref.at[pl.ds(rbase, CI)], dstc4.at[b], edg.at[b])

    def _wait_edge(b):
        pltpu.make_async_copy(src_ref.at[pl.ds(0, C)], src4.at[b], edg.at[b]).wait()
        pltpu.make_async_copy(et_ref.at[pl.ds(0, C)], et4.at[b], edg.at[b]).wait()
        pltpu.make_async_copy(norm_ref.at[pl.ds(0, C)], norm4.at[b], edg.at[b]).wait()
        pltpu.make_async_copy(dst_ref.at[pl.ds(0, CI)], dstc4.at[b], edg.at[b]).wait()

    def _scale_fire_scatter(ci):
        pr = lax.bitwise_and(ci - 1, 1)
        pb = lax.bitwise_and(ci - 1, 3)
        pltpu.make_async_copy(table_ref.at[pl.ds(0, C)], rows_v.at[pr],
                              gth.at[pr]).wait()

        @plsc.parallel_loop(0, C // L, 1, unroll=2)
        def _scale(g):
            nv = norm4[pb, pl.ds(g * L, L)]
            rr = g * L
            for j in range(L):
                rows_v[pr, rr + j, :] = rows_v[pr, rr + j, :] * _splat(nv, j)

        for j in range(CI):
            pltpu.async_copy(rows_v.at[pr, pl.ds(j * 128, 128)],
                             acc.at[dstc4.at[pb, j]], sct.at[pr], add=True)

    def _gidx_for(ci):
        b = lax.bitwise_and(ci, 3)
        _wait_edge(b)

        def _gidx(g, _):
            s = src4[b, pl.ds(g * L, L)]
            e = et4[b, pl.ds(g * L, L)]
            gidx4[b, pl.ds(g * L, L)] = (e * IN_C + s) * 2 + h
            return 0
        lax.fori_loop(0, C // L, _gidx, 0)

    def _chunk(ci, _):
        b = lax.bitwise_and(ci, 3)
        r = lax.bitwise_and(ci, 1)

        def _wait_sct():
            pltpu.make_async_copy(table_ref.at[pl.ds(0, C)], rows_v.at[r],
                                  sct.at[r]).wait()
        pl.when(ci >= 2)(_wait_sct)

        for j in range(CI):
            pltpu.async_copy(table_ref.at[gidx4.at[b, pl.ds(j * 128, 128)]],
                             rows_v.at[r, pl.ds(j * 128, 128)], gth.at[r])

        pl.when(ci >= 1)(lambda: _scale_fire_scatter(ci))

        pl.when(ci + 1 < n_t)(lambda: _gidx_for(ci + 1))

        nb = lax.bitwise_and(ci + 2, 3)
        pl.when(ci + 2 < n_t)(lambda: _fire_edge(ci + 2, nb))
        return 0

    _fire_edge(0, 0)
    _fire_edge(1, 1)
    _gidx_for(0)
    lax.fori_loop(0, n_t, _chunk, 0)
    _scale_fire_scatter(n_t)
    for rr in range(2):
        pltpu.make_async_copy(table_ref.at[pl.ds(0, C)], rows_v.at[rr],
                              sct.at[rr]).wait()
    plsc.subcore_barrier()

    for i in range(RPT // C):
        pltpu.sync_copy(acc.at[pl.ds(r0 + i * C, C)],
                        out_ref.at[h, pl.ds(r0 + i * C, C)])
    if rem:
        pltpu.sync_copy(acc.at[pl.ds(r0 + (RPT // C) * C, rem)],
                        out_ref.at[h, pl.ds(r0 + (RPT // C) * C, rem)])


def _sc_aggregate(table2, src, et, dst2, norm):
    fn = pl.kernel(
        _sc_body,
        out_type=jax.ShapeDtypeStruct((NC, N_PAD, L), jnp.float32),
        mesh=plsc.VectorSubcoreMesh(core_axis_name="c", subcore_axis_name="s"),
        compiler_params=pltpu.CompilerParams(use_tc_tiling_on_sc=False),
        scratch_types=[
            pltpu.VMEM((4, C), jnp.int32),
            pltpu.VMEM((4, C), jnp.int32),
            pltpu.VMEM((4, C), jnp.int32),
            pltpu.VMEM((4, CI, 128), jnp.int32),
            pltpu.VMEM((4, C), jnp.float32),
            pltpu.VMEM((2, C, L), jnp.float32),
            pltpu.VMEM_SHARED((N_PAD, L), jnp.float32),
            pltpu.SemaphoreType.DMA((4,)),
            pltpu.SemaphoreType.DMA((2,)),
            pltpu.SemaphoreType.DMA((2,)),
        ],
    )
    return fn(table2, src, et, dst2, norm)


def _tc_body(a_ref, b_ref, wu_ref, wi_ref, o_ref):
    i = pl.program_id(0)
    a = jnp.maximum(a_ref[0], 0.0)
    b = jnp.maximum(b_ref[0], 0.0)
    x = jnp.concatenate([a, b], axis=1)
    ou = lax.dot_general(x, wu_ref[...], (((1,), (0,)), ((), ())),
                         preferred_element_type=jnp.float32)
    oi = lax.dot_general(x, wi_ref[...], (((1,), (0,)), ((), ())),
                         preferred_element_type=jnp.float32)
    rows = lax.broadcasted_iota(jnp.int32, (BF, 8 * OUT_C), 0) + i * BF
    o = jnp.where(rows < N_USER // 8, ou, oi)
    o_ref[...] = jnp.maximum(o, 0.0)


BF = 736


def _tc_dense(agg_flat, w_u, w_i):
    eye8 = jnp.eye(8, dtype=jnp.float32)
    wu = jnp.concatenate([jnp.kron(eye8, w_u[:16]), jnp.kron(eye8, w_u[16:])])
    wi = jnp.concatenate([jnp.kron(eye8, w_i[:16]), jnp.kron(eye8, w_i[16:])])
    n_flat = N_PAD // 8
    grid = n_flat // BF
    return pl.pallas_call(
        _tc_body,
        grid=(grid,),
        in_specs=[
            pl.BlockSpec((1, BF, 128), lambda i: (0, i, 0)),
            pl.BlockSpec((1, BF, 128), lambda i: (1, i, 0)),
            pl.BlockSpec((256, 8 * OUT_C), lambda i: (0, 0)),
            pl.BlockSpec((256, 8 * OUT_C), lambda i: (0, 0)),
        ],
        out_specs=pl.BlockSpec((BF, 8 * OUT_C), lambda i: (i, 0)),
        out_shape=jax.ShapeDtypeStruct((n_flat, 8 * OUT_C), jnp.float32),
    )(agg_flat, agg_flat, wu, wi)


def kernel(x, edge_index, edge_type, edge_norm, rgc_weight, dense_w_u, dense_w_i):
    src = edge_index[0].astype(jnp.int32)
    et = edge_type.astype(jnp.int32)
    norm = edge_norm.astype(jnp.float32)
    dst2 = edge_index[1].astype(jnp.int32).reshape(E_REAL // 128, 128)
    table2 = rgc_weight.reshape(-1, L)

    agg = _sc_aggregate(table2, src, et, dst2, norm)
    agg_flat = agg.reshape(NC, N_PAD // 8, 128)

    out = _tc_dense(agg_flat, dense_w_u, dense_w_i)
    out = out.reshape(N_PAD, OUT_C)
    return (out[:N_USER], out[N_USER:N_NODES])
